# Initial kernel scaffold; baseline (speedup 1.0000x reference)
import jax
import jax.numpy as jnp
from jax.experimental import pallas as pl


def kernel(input_ids, emb_table, out_weight):
    raise NotImplementedError("write your pallas kernel here")


# speedup vs baseline: 3.6128x; 3.6128x over previous
import functools

import jax
import jax.numpy as jnp
from jax import lax
from jax.experimental import pallas as pl
from jax.experimental.pallas import tpu as pltpu
from jax.experimental.pallas import tpu_sc as plsc

_NUM_CORES = 2
_NUM_SUBCORES = 16
_NUM_WORKERS = _NUM_CORES * _NUM_SUBCORES
_CHUNK = 128


def _matmul_body(emb_ref, w_ref, g_ref):
    g_ref[...] = lax.dot_general(
        emb_ref[...], w_ref[...],
        dimension_numbers=(((1,), (1,)), ((), ())),
        preferred_element_type=jnp.float32,
    )


def _logit_table(emb_table, out_weight):
    v = emb_table.shape[0]
    return pl.pallas_call(
        _matmul_body,
        out_shape=jax.ShapeDtypeStruct((v, v), jnp.float32),
    )(emb_table, out_weight)


@functools.partial(jax.jit, static_argnames=("n_per_w", "n_chunks"))
def _sc_gather(g, ids3, n_per_w, n_chunks):
    v = g.shape[1]
    mesh = plsc.VectorSubcoreMesh(
        core_axis_name="c", subcore_axis_name="s",
        num_cores=_NUM_CORES, num_subcores=_NUM_SUBCORES,
    )

    @functools.partial(
        pl.kernel,
        mesh=mesh,
        compiler_params=pltpu.CompilerParams(use_tc_tiling_on_sc=False),
        out_type=jax.ShapeDtypeStruct((_NUM_WORKERS, n_per_w, v), jnp.float32),
        scratch_types=[
            pltpu.VMEM((n_chunks, _CHUNK), jnp.int32),
            pltpu.VMEM((n_per_w, v), jnp.float32),
            pltpu.SemaphoreType.DMA,
        ],
    )
    def gather_kernel(g_hbm, ids_hbm, out_hbm, idx_v, rows_v, sem):
        wid = lax.axis_index("s") * _NUM_CORES + lax.axis_index("c")
        pltpu.sync_copy(ids_hbm.at[wid], idx_v)
        copies = []
        for j in range(n_chunks):
            copies.append(pltpu.async_copy(
                g_hbm.at[idx_v.at[j]],
                rows_v.at[pl.ds(j * _CHUNK, _CHUNK)],
                sem,
            ))
        for c in copies:
            c.wait()
        pltpu.sync_copy(rows_v, out_hbm.at[wid])

    return gather_kernel(g, ids3)


def kernel(input_ids, emb_table, out_weight):
    b, s = input_ids.shape
    v = emb_table.shape[0]
    n = b * s
    n_per_w = n // _NUM_WORKERS
    n_chunks = n_per_w // _CHUNK

    g = _logit_table(emb_table, out_weight)
    ids3 = input_ids.reshape(_NUM_WORKERS, n_chunks, _CHUNK).astype(jnp.int32)
    out = _sc_gather(g, ids3, n_per_w, n_chunks)
    return out.reshape(b, s, v)


# speedup vs baseline: 3.7328x; 1.0332x over previous
import functools

import jax
import jax.numpy as jnp
from jax import lax
from jax.experimental import pallas as pl
from jax.experimental.pallas import tpu as pltpu
from jax.experimental.pallas import tpu_sc as plsc

_NUM_CORES = 2
_NUM_SUBCORES = 16
_NUM_WORKERS = _NUM_CORES * _NUM_SUBCORES
_LANES = 16


def _matmul_body(emb_ref, w_ref, g_ref):
    g_ref[...] = lax.dot_general(
        emb_ref[...], w_ref[...],
        dimension_numbers=(((1,), (1,)), ((), ())),
        preferred_element_type=jnp.float32,
    )


def _logit_table(emb_table, out_weight):
    v = emb_table.shape[0]
    return pl.pallas_call(
        _matmul_body,
        out_shape=jax.ShapeDtypeStruct((v, v), jnp.float32),
    )(emb_table, out_weight)


@functools.partial(jax.jit, static_argnames=("n_per_w", "vocab"))
def _sc_expand(g_flat, ids2, n_per_w, vocab):
    mesh = plsc.VectorSubcoreMesh(
        core_axis_name="c", subcore_axis_name="s",
        num_cores=_NUM_CORES, num_subcores=_NUM_SUBCORES,
    )
    n_blocks = n_per_w // _LANES

    @functools.partial(
        pl.kernel,
        mesh=mesh,
        compiler_params=pltpu.CompilerParams(
            use_tc_tiling_on_sc=False, needs_layout_passes=False),
        out_type=jax.ShapeDtypeStruct((_NUM_WORKERS, n_per_w * vocab), jnp.float32),
        scratch_types=[
            pltpu.VMEM((vocab * vocab,), jnp.float32),
            pltpu.VMEM((n_per_w,), jnp.int32),
            pltpu.VMEM((n_per_w * vocab,), jnp.float32),
        ],
    )
    def expand_kernel(g_hbm, ids_hbm, out_hbm, g_v, idx_v, out_v):
        wid = lax.axis_index("s") * _NUM_CORES + lax.axis_index("c")
        pltpu.sync_copy(g_hbm, g_v)
        pltpu.sync_copy(ids_hbm.at[wid], idx_v)

        lane = lax.iota(jnp.int32, _LANES)

        def block_body(b, _):
            base = b * _LANES
            idx = idx_v[pl.ds(base, _LANES)]
            gbase = idx * vocab
            obase = (base + lane) * vocab
            for col in range(vocab):
                vals = plsc.load_gather(g_v, [gbase + col])
                plsc.store_scatter(out_v, [obase + col], vals)
            return _

        lax.fori_loop(0, n_blocks, block_body, 0)
        pltpu.sync_copy(out_v, out_hbm.at[wid])

    return expand_kernel(g_flat, ids2)


def kernel(input_ids, emb_table, out_weight):
    b, s = input_ids.shape
    vocab = emb_table.shape[0]
    n = b * s
    n_per_w = n // _NUM_WORKERS

    g = _logit_table(emb_table, out_weight)
    ids2 = input_ids.reshape(_NUM_WORKERS, n_per_w).astype(jnp.int32)
    out = _sc_expand(g.reshape(vocab * vocab), ids2, n_per_w, vocab)
    return out.reshape(b, s, vocab)


# speedup vs baseline: 5.6960x; 1.5259x over previous
import functools

import jax
import jax.numpy as jnp
from jax import lax
from jax.experimental import pallas as pl
from jax.experimental.pallas import tpu as pltpu
from jax.experimental.pallas import tpu_sc as plsc

_NUM_CORES = 2
_NUM_SUBCORES = 16
_NUM_WORKERS = _NUM_CORES * _NUM_SUBCORES
_LANES = 16


def _matmul_body(emb_ref, w_ref, g_ref):
    g_ref[...] = lax.dot_general(
        emb_ref[...], w_ref[...],
        dimension_numbers=(((1,), (1,)), ((), ())),
        preferred_element_type=jnp.float32,
    )


def _logit_table(emb_table, out_weight):
    v = emb_table.shape[0]
    return pl.pallas_call(
        _matmul_body,
        out_shape=jax.ShapeDtypeStruct((v, v), jnp.float32),
    )(emb_table, out_weight)


@functools.partial(jax.jit, static_argnames=("n_per_w", "vocab"))
def _sc_expand(g_flat, ids2, n_per_w, vocab):
    mesh = plsc.VectorSubcoreMesh(
        core_axis_name="c", subcore_axis_name="s",
        num_cores=_NUM_CORES, num_subcores=_NUM_SUBCORES,
    )
    n_blocks = n_per_w // _LANES

    @functools.partial(
        pl.kernel,
        mesh=mesh,
        compiler_params=pltpu.CompilerParams(
            use_tc_tiling_on_sc=False, needs_layout_passes=False),
        out_type=jax.ShapeDtypeStruct((_NUM_WORKERS, n_per_w * vocab), jnp.float32),
        scratch_types=[
            pltpu.VMEM((vocab * vocab,), jnp.float32),
            pltpu.VMEM((n_per_w,), jnp.int32),
            pltpu.VMEM((n_per_w * vocab,), jnp.float32),
        ],
    )
    def expand_kernel(g_hbm, ids_hbm, out_hbm, g_v, idx_v, out_v):
        wid = lax.axis_index("s") * _NUM_CORES + lax.axis_index("c")
        pltpu.sync_copy(g_hbm, g_v)
        pltpu.sync_copy(ids_hbm.at[wid], idx_v)

        def block_body(b, _):
            base = b * _LANES
            idx = idx_v[pl.ds(base, _LANES)] * vocab
            for t in range(_LANES):
                tok = base + t
                goff = idx[t]
                ooff = tok * vocab
                for h in range(vocab // _LANES):
                    out_v[pl.ds(ooff + h * _LANES, _LANES)] = (
                        g_v[pl.ds(goff + h * _LANES, _LANES)])
            return _

        lax.fori_loop(0, n_blocks, block_body, 0)
        pltpu.sync_copy(out_v, out_hbm.at[wid])

    return expand_kernel(g_flat, ids2)


def kernel(input_ids, emb_table, out_weight):
    b, s = input_ids.shape
    vocab = emb_table.shape[0]
    n = b * s
    n_per_w = n // _NUM_WORKERS

    g = _logit_table(emb_table, out_weight)
    ids2 = input_ids.reshape(_NUM_WORKERS, n_per_w).astype(jnp.int32)
    out = _sc_expand(g.reshape(vocab * vocab), ids2, n_per_w, vocab)
    return out.reshape(b, s, vocab)


# speedup vs baseline: 6.4365x; 1.1300x over previous
import functools

import jax
import jax.numpy as jnp
from jax import lax
from jax.experimental import pallas as pl
from jax.experimental.pallas import tpu as pltpu
from jax.experimental.pallas import tpu_sc as plsc

_NUM_CORES = 2
_NUM_SUBCORES = 16
_NUM_WORKERS = _NUM_CORES * _NUM_SUBCORES
_LANES = 16


def _matmul_body(emb_ref, w_ref, g_ref):
    g_ref[...] = lax.dot_general(
        emb_ref[...], w_ref[...],
        dimension_numbers=(((1,), (1,)), ((), ())),
        preferred_element_type=jnp.float32,
    )


def _logit_table(emb_table, out_weight):
    v = emb_table.shape[0]
    return pl.pallas_call(
        _matmul_body,
        out_shape=jax.ShapeDtypeStruct((v, v), jnp.float32),
    )(emb_table, out_weight)


@functools.partial(jax.jit, static_argnames=("n_per_w", "vocab"))
def _sc_expand(g_flat, ids2, n_per_w, vocab):
    mesh = plsc.VectorSubcoreMesh(
        core_axis_name="c", subcore_axis_name="s",
        num_cores=_NUM_CORES, num_subcores=_NUM_SUBCORES,
    )
    n_blocks = n_per_w // _LANES

    @functools.partial(
        pl.kernel,
        mesh=mesh,
        compiler_params=pltpu.CompilerParams(
            use_tc_tiling_on_sc=False, needs_layout_passes=False),
        out_type=jax.ShapeDtypeStruct((_NUM_WORKERS, n_per_w * vocab), jnp.float32),
        scratch_types=[
            pltpu.VMEM((vocab * vocab,), jnp.float32),
            pltpu.VMEM((n_per_w,), jnp.int32),
            pltpu.VMEM((n_per_w * vocab,), jnp.float32),
        ],
    )
    def expand_kernel(g_hbm, ids_hbm, out_hbm, g_v, idx_v, out_v):
        wid = lax.axis_index("s") * _NUM_CORES + lax.axis_index("c")
        pltpu.sync_copy(g_hbm, g_v)
        pltpu.sync_copy(ids_hbm.at[wid], idx_v)

        @plsc.parallel_loop(0, n_blocks)
        def block_body(b):
            base = b * _LANES
            idx = idx_v[pl.ds(base, _LANES)] * vocab
            for t in range(_LANES):
                tok = base + t
                goff = idx[t]
                ooff = tok * vocab
                for h in range(vocab // _LANES):
                    out_v[pl.ds(ooff + h * _LANES, _LANES)] = (
                        g_v[pl.ds(goff + h * _LANES, _LANES)])

        pltpu.sync_copy(out_v, out_hbm.at[wid])

    return expand_kernel(g_flat, ids2)


def kernel(input_ids, emb_table, out_weight):
    b, s = input_ids.shape
    vocab = emb_table.shape[0]
    n = b * s
    n_per_w = n // _NUM_WORKERS

    g = _logit_table(emb_table, out_weight)
    ids2 = input_ids.reshape(_NUM_WORKERS, n_per_w).astype(jnp.int32)
    out = _sc_expand(g.reshape(vocab * vocab), ids2, n_per_w, vocab)
    return out.reshape(b, s, vocab)
